# baseline (speedup 1.0000x reference)
import functools

import jax
import jax.numpy as jnp
from jax import lax
from jax.experimental import pallas as pl
from jax.experimental.pallas import tpu as pltpu
from jax.experimental.pallas import tpu_sc as plsc

_V, _D, _H, _C, _B, _T = 100000, 128, 300, 1000, 1024, 32
_HP = 384
_CP = 1024


def _sc_gather(emb, idx_flat):
    n = idx_flat.shape[0]
    info = plsc.get_sparse_core_info()
    nw = info.num_cores * info.num_subcores
    b_per_w = n // nw
    ch = min(256, b_per_w)
    n_ch = b_per_w // ch
    mesh = plsc.VectorSubcoreMesh(core_axis_name="c", subcore_axis_name="s")

    @functools.partial(
        pl.kernel,
        mesh=mesh,
        out_type=jax.ShapeDtypeStruct((n, _D), jnp.float32),
        scratch_types=[
            pltpu.VMEM((b_per_w,), jnp.int32),
            pltpu.VMEM((2, ch, _D), jnp.float32),
            pltpu.SemaphoreType.DMA,
            pltpu.SemaphoreType.DMA,
            pltpu.SemaphoreType.DMA,
            pltpu.SemaphoreType.DMA,
        ],
    )
    def gather_kernel(table_hbm, idx_hbm, out_hbm, idx_v, rows_v,
                      gs0, gs1, os0, os1):
        wid = lax.axis_index("s") * info.num_cores + lax.axis_index("c")
        base = wid * b_per_w
        pltpu.sync_copy(idx_hbm.at[pl.ds(base, b_per_w)], idx_v)
        gsems, osems = (gs0, gs1), (os0, os1)

        def start_get(c):
            return pltpu.async_copy(
                table_hbm.at[idx_v.at[pl.ds(c * ch, ch)]],
                rows_v.at[c % 2], gsems[c % 2])

        def start_put(c):
            return pltpu.async_copy(
                rows_v.at[c % 2], out_hbm.at[pl.ds(base + c * ch, ch)],
                osems[c % 2])

        gets, puts = [None] * n_ch, [None] * n_ch
        gets[0] = start_get(0)
        if n_ch > 1:
            gets[1] = start_get(1)
        for c in range(n_ch):
            gets[c].wait()
            puts[c] = start_put(c)
            if c + 2 < n_ch:
                puts[c].wait()
                gets[c + 2] = start_get(c + 2)
        for c in range(max(0, n_ch - 2), n_ch):
            puts[c].wait()

    return gather_kernel(emb, idx_flat)


def _tc_gru_chunk(x_tm, h_in, w_all, wout, bout, final):
    t_len = x_tm.shape[0]
    bf = jnp.bfloat16

    def body(x_ref, h0_ref, wall_ref, wout_ref, bout_ref, out_ref):
        w_a = wall_ref[...]

        def step(t, h):
            x_t = x_ref[t].astype(bf)
            h_b = h.astype(bf)
            xh = jnp.concatenate([x_t, h_b], axis=1)
            g = jnp.dot(xh, w_a, preferred_element_type=jnp.float32)
            r = jax.nn.sigmoid(g[:, :_HP])
            z = jax.nn.sigmoid(g[:, _HP:2 * _HP])
            n = jnp.tanh(g[:, 2 * _HP:3 * _HP] + r * g[:, 3 * _HP:])
            return n + z * (h - n)

        h = lax.fori_loop(0, t_len, step, h0_ref[...], unroll=2)
        if final:
            out_ref[...] = (
                jnp.dot(h, wout_ref[...], preferred_element_type=jnp.float32)
                + bout_ref[...])
        else:
            out_ref[...] = h

    out_shape = (_B, _CP) if final else (_B, _HP)
    return pl.pallas_call(
        body,
        out_shape=jax.ShapeDtypeStruct(out_shape, jnp.float32),
    )(x_tm, h_in, w_all, wout, bout)


def _pad_gates(w, k):
    parts = jnp.split(w, k, axis=-1)
    pad = [(0, 0)] * (w.ndim - 1) + [(0, _HP - _H)]
    return jnp.concatenate([jnp.pad(p, pad) for p in parts], axis=-1)


def kernel(sequence, emb, W_ih, W_hh, b_ih, b_hh, W_out, b_out):
    idx = jnp.asarray(sequence, jnp.int32).T.reshape(-1)

    bf = jnp.bfloat16
    zpad = jnp.zeros((_D, _HP), jnp.float32)
    zpad_h = jnp.zeros((_HP, _HP), jnp.float32)
    x_rows = jnp.concatenate(
        [_pad_gates(W_ih[:, :2 * _H], 2), _pad_gates(W_ih[:, 2 * _H:], 1),
         zpad], axis=1)
    bias_row = jnp.concatenate([
        _pad_gates(b_ih[:2 * _H] + b_hh[:2 * _H], 2)
        .at[_HP + _H].set(20.0),
        _pad_gates(b_ih[2 * _H:], 1),
        _pad_gates(b_hh[2 * _H:], 1),
    ])
    h_rows = jnp.concatenate(
        [jnp.pad(_pad_gates(W_hh[:, :2 * _H], 2), ((0, _HP - _H), (0, 0))),
         zpad_h,
         jnp.pad(_pad_gates(W_hh[:, 2 * _H:], 1), ((0, _HP - _H), (0, 0)))],
        axis=1).at[_H].set(bias_row)
    w_all = jnp.concatenate([x_rows, h_rows], axis=0).astype(bf)
    wout = jnp.pad(W_out, ((0, _HP - _H), (0, _CP - _C)))
    bout = jnp.pad(b_out, ((0, _CP - _C)))[None, :]

    x = _sc_gather(emb, idx).reshape(_T, _B, _D)
    h0 = jnp.zeros((_B, _HP), jnp.float32).at[:, _H].set(1.0)
    logits = _tc_gru_chunk(x, h0, w_all, wout, bout, final=True)
    return logits[:, :_C]
